# baseline (speedup 1.0000x reference)
import jax
import jax.numpy as jnp
from jax.experimental import pallas as pl
from jax.experimental.pallas import tpu as pltpu

_THRESHOLD = 0.3
_PENALTY_WEIGHT = 0.05

_B, _H, _W, _C = 8, 384, 384, 96
_BH = 64
_NH = _H // _BH
_NSTEPS = _B * _NH


def _bounds(vals, idx, thr, size):
    mask = vals > thr
    mn = jnp.min(jnp.min(jnp.where(mask, idx, size), axis=0, keepdims=True),
                 axis=1, keepdims=True)
    mx = jnp.max(jnp.max(jnp.where(mask, idx, -1), axis=0, keepdims=True),
                 axis=1, keepdims=True)
    empty = mn == size
    mn = jnp.where(empty, 0, mn)
    mx = jnp.where(empty, 1, mx)
    return mn.astype(jnp.float32), mx.astype(jnp.float32)


def _fused(xp_ref, xt_ref, out_ref, zp, zt, rowp, rowt, colp, colt):
    b = pl.program_id(0)
    h = pl.program_id(1)
    i = b * _NH + h
    xp = xp_ref[0]
    xt = xt_ref[0]

    rowp[i, :] = jnp.max(jnp.max(xp, axis=1), axis=1)
    rowt[i, :] = jnp.max(jnp.max(xt, axis=1), axis=1)
    zp_part = jnp.max(xp, axis=0)
    zt_part = jnp.max(xt, axis=0)

    @pl.when(h == 0)
    def _():
        zp[...] = zp_part
        zt[...] = zt_part

    @pl.when(h != 0)
    def _():
        zp[...] = jnp.maximum(zp[...], zp_part)
        zt[...] = jnp.maximum(zt[...], zt_part)

    @pl.when(h == _NH - 1)
    def _():
        colp[b, :] = jnp.max(zp[...], axis=0)
        colt[b, :] = jnp.max(zt[...], axis=0)

    @pl.when(i == _NSTEPS - 1)
    def _():
        yidx = (jax.lax.broadcasted_iota(jnp.int32, (_NH, _BH), 0) * _BH
                + jax.lax.broadcasted_iota(jnp.int32, (_NH, _BH), 1))
        xidx = jax.lax.broadcasted_iota(jnp.int32, (1, _W), 1)
        total = jnp.zeros((1, 1), jnp.float32)
        for bb in range(_B):
            rp = rowp[bb * _NH:(bb + 1) * _NH, :]
            rt = rowt[bb * _NH:(bb + 1) * _NH, :]
            cp = colp[bb:bb + 1, :]
            ct = colt[bb:bb + 1, :]
            p_y1, p_y2 = _bounds(rp, yidx, _THRESHOLD, _H)
            p_x1, p_x2 = _bounds(cp, xidx, _THRESHOLD, _W)
            t_y1, t_y2 = _bounds(rt, yidx, 0.5, _H)
            t_x1, t_x2 = _bounds(ct, xidx, 0.5, _W)

            pred_area = (p_y2 - p_y1 + 1.0) * (p_x2 - p_x1 + 1.0)
            true_area = (t_y2 - t_y1 + 1.0) * (t_x2 - t_x1 + 1.0)
            area_penalty = (jnp.maximum(pred_area - true_area, 0.0)
                            / (true_area + 1.0))
            dy = (p_y1 + p_y2 - t_y1 - t_y2) * 0.5
            dx = (p_x1 + p_x2 - t_x1 - t_x2) * 0.5
            center_offset = jnp.sqrt(dy * dy + dx * dx) / 20.0
            total = total + area_penalty + center_offset
        out_ref[...] = (_PENALTY_WEIGHT / _B) * total


def kernel(prediction_probs, expected_onehot):
    xp = jnp.transpose(prediction_probs, (0, 1, 3, 2))
    xt = jnp.transpose(expected_onehot, (0, 1, 3, 2))
    out = pl.pallas_call(
        _fused,
        grid=(_B, _NH),
        in_specs=[
            pl.BlockSpec((1, _BH, _C, _W), lambda b, h: (b, h, 0, 0)),
            pl.BlockSpec((1, _BH, _C, _W), lambda b, h: (b, h, 0, 0)),
        ],
        out_specs=pl.BlockSpec((1, 1), lambda b, h: (0, 0)),
        out_shape=jax.ShapeDtypeStruct((1, 1), jnp.float32),
        scratch_shapes=[
            pltpu.VMEM((_C, _W), jnp.float32),
            pltpu.VMEM((_C, _W), jnp.float32),
            pltpu.VMEM((_NSTEPS, _BH), jnp.float32),
            pltpu.VMEM((_NSTEPS, _BH), jnp.float32),
            pltpu.VMEM((_B, _W), jnp.float32),
            pltpu.VMEM((_B, _W), jnp.float32),
        ],
    )(xp, xt)
    return out[0, 0]


# speedup vs baseline: 1.0017x; 1.0017x over previous
import jax
import jax.numpy as jnp
from jax.experimental import pallas as pl
from jax.experimental.pallas import tpu as pltpu

_THRESHOLD = 0.3
_PENALTY_WEIGHT = 0.05

_B, _H, _W, _C = 8, 384, 384, 96
_BH = 48
_NH = _H // _BH
_NSTEPS = _B * _NH


def _bounds(vals, idx, thr, size):
    mask = vals > thr
    mn = jnp.min(jnp.min(jnp.where(mask, idx, size), axis=0, keepdims=True),
                 axis=1, keepdims=True)
    mx = jnp.max(jnp.max(jnp.where(mask, idx, -1), axis=0, keepdims=True),
                 axis=1, keepdims=True)
    empty = mn == size
    mn = jnp.where(empty, 0, mn)
    mx = jnp.where(empty, 1, mx)
    return mn.astype(jnp.float32), mx.astype(jnp.float32)


def _fused(xp_ref, xt_ref, out_ref, zp, zt, rowp, rowt, colp, colt):
    b = pl.program_id(0)
    h = pl.program_id(1)
    i = b * _NH + h
    xp = xp_ref[0]
    xt = xt_ref[0]

    rowp[i, :] = jnp.max(jnp.max(xp, axis=1), axis=1)
    rowt[i, :] = jnp.max(jnp.max(xt, axis=1), axis=1)
    zp_part = jnp.max(xp, axis=0)
    zt_part = jnp.max(xt, axis=0)

    @pl.when(h == 0)
    def _():
        zp[...] = zp_part
        zt[...] = zt_part

    @pl.when(h != 0)
    def _():
        zp[...] = jnp.maximum(zp[...], zp_part)
        zt[...] = jnp.maximum(zt[...], zt_part)

    @pl.when(h == _NH - 1)
    def _():
        colp[b, :] = jnp.max(zp[...], axis=0)
        colt[b, :] = jnp.max(zt[...], axis=0)

    @pl.when(i == _NSTEPS - 1)
    def _():
        yidx = (jax.lax.broadcasted_iota(jnp.int32, (_NH, _BH), 0) * _BH
                + jax.lax.broadcasted_iota(jnp.int32, (_NH, _BH), 1))
        xidx = jax.lax.broadcasted_iota(jnp.int32, (1, _W), 1)
        total = jnp.zeros((1, 1), jnp.float32)
        for bb in range(_B):
            rp = rowp[bb * _NH:(bb + 1) * _NH, :]
            rt = rowt[bb * _NH:(bb + 1) * _NH, :]
            cp = colp[bb:bb + 1, :]
            ct = colt[bb:bb + 1, :]
            p_y1, p_y2 = _bounds(rp, yidx, _THRESHOLD, _H)
            p_x1, p_x2 = _bounds(cp, xidx, _THRESHOLD, _W)
            t_y1, t_y2 = _bounds(rt, yidx, 0.5, _H)
            t_x1, t_x2 = _bounds(ct, xidx, 0.5, _W)

            pred_area = (p_y2 - p_y1 + 1.0) * (p_x2 - p_x1 + 1.0)
            true_area = (t_y2 - t_y1 + 1.0) * (t_x2 - t_x1 + 1.0)
            area_penalty = (jnp.maximum(pred_area - true_area, 0.0)
                            / (true_area + 1.0))
            dy = (p_y1 + p_y2 - t_y1 - t_y2) * 0.5
            dx = (p_x1 + p_x2 - t_x1 - t_x2) * 0.5
            center_offset = jnp.sqrt(dy * dy + dx * dx) / 20.0
            total = total + area_penalty + center_offset
        out_ref[...] = (_PENALTY_WEIGHT / _B) * total


def kernel(prediction_probs, expected_onehot):
    xp = jnp.transpose(prediction_probs, (0, 1, 3, 2))
    xt = jnp.transpose(expected_onehot, (0, 1, 3, 2))
    out = pl.pallas_call(
        _fused,
        grid=(_B, _NH),
        in_specs=[
            pl.BlockSpec((1, _BH, _C, _W), lambda b, h: (b, h, 0, 0)),
            pl.BlockSpec((1, _BH, _C, _W), lambda b, h: (b, h, 0, 0)),
        ],
        out_specs=pl.BlockSpec((1, 1), lambda b, h: (0, 0)),
        out_shape=jax.ShapeDtypeStruct((1, 1), jnp.float32),
        scratch_shapes=[
            pltpu.VMEM((_C, _W), jnp.float32),
            pltpu.VMEM((_C, _W), jnp.float32),
            pltpu.VMEM((_NSTEPS, _BH), jnp.float32),
            pltpu.VMEM((_NSTEPS, _BH), jnp.float32),
            pltpu.VMEM((_B, _W), jnp.float32),
            pltpu.VMEM((_B, _W), jnp.float32),
        ],
    )(xp, xt)
    return out[0, 0]
